# baseline (device time: 215565 ns/iter reference)
import jax
import jax.numpy as jnp
from jax import lax
from jax.experimental import pallas as pl
from jax.experimental.pallas import tpu as pltpu

N_DEV = 16


def kernel(x, w_mat, scale_x, scale_w):
    m_per, k = x.shape
    _, n_per = w_mat.shape
    m_full = N_DEV * m_per

    scale = (scale_x[0] * scale_w[0]).reshape(1, 1).astype(jnp.float32)

    def body(x_ref, w_ref, scale_ref, out_ref, xfull_ref, send_sems, recv_sems):
        my = lax.axis_index("i")
        left = lax.rem(my + N_DEV - 1, N_DEV)
        right = lax.rem(my + 1, N_DEV)

        barrier_sem = pltpu.get_barrier_semaphore()
        for nbr in (left, right):
            pl.semaphore_signal(
                barrier_sem, inc=1,
                device_id=(nbr,), device_id_type=pl.DeviceIdType.MESH,
            )
        pl.semaphore_wait(barrier_sem, 2)

        s = scale_ref[0, 0]

        def matmul_store(origin):
            chunk = xfull_ref[pl.ds(origin * m_per, m_per), :]
            acc = lax.dot_general(
                chunk, w_ref[...],
                dimension_numbers=(((1,), (0,)), ((), ())),
                preferred_element_type=jnp.int32,
            )
            out_ref[pl.ds(origin * m_per, m_per), :] = acc.astype(jnp.float32) * s

        xfull_ref[pl.ds(my * m_per, m_per), :] = x_ref[...]
        matmul_store(my)

        for h in range(N_DEV - 1):
            send_origin = lax.rem(my - h + 2 * N_DEV, N_DEV)
            recv_origin = lax.rem(my - h - 1 + 2 * N_DEV, N_DEV)
            rdma = pltpu.make_async_remote_copy(
                src_ref=xfull_ref.at[pl.ds(send_origin * m_per, m_per)],
                dst_ref=xfull_ref.at[pl.ds(send_origin * m_per, m_per)],
                send_sem=send_sems.at[h],
                recv_sem=recv_sems.at[h],
                device_id=(right,),
                device_id_type=pl.DeviceIdType.MESH,
            )
            rdma.start()
            rdma.wait()
            matmul_store(recv_origin)

    return pl.pallas_call(
        body,
        out_shape=jax.ShapeDtypeStruct((m_full, n_per), jnp.float32),
        in_specs=[
            pl.BlockSpec(memory_space=pltpu.VMEM),
            pl.BlockSpec(memory_space=pltpu.VMEM),
            pl.BlockSpec(memory_space=pltpu.SMEM),
        ],
        out_specs=pl.BlockSpec(memory_space=pltpu.VMEM),
        scratch_shapes=[
            pltpu.VMEM((m_full, k), jnp.int8),
            pltpu.SemaphoreType.DMA((N_DEV - 1,)),
            pltpu.SemaphoreType.DMA((N_DEV - 1,)),
        ],
        compiler_params=pltpu.CompilerParams(collective_id=0),
    )(x, w_mat, scale)


# device time: 113355 ns/iter; 1.9017x vs baseline; 1.9017x over previous
import jax
import jax.numpy as jnp
from jax import lax
from jax.experimental import pallas as pl
from jax.experimental.pallas import tpu as pltpu

N_DEV = 16
R_HOPS = N_DEV // 2
L_HOPS = N_DEV - 1 - R_HOPS


def kernel(x, w_mat, scale_x, scale_w):
    m_per, k = x.shape
    _, n_per = w_mat.shape
    m_full = N_DEV * m_per

    scale = (scale_x[0] * scale_w[0]).reshape(1, 1).astype(jnp.float32)

    def body(x_ref, w_ref, scale_ref, out_ref,
             xfull_ref, r_send_sems, r_recv_sems, l_send_sems, l_recv_sems):
        my = lax.axis_index("i")
        left = lax.rem(my + N_DEV - 1, N_DEV)
        right = lax.rem(my + 1, N_DEV)

        barrier_sem = pltpu.get_barrier_semaphore()
        for nbr in (left, right):
            pl.semaphore_signal(
                barrier_sem, inc=1,
                device_id=(nbr,), device_id_type=pl.DeviceIdType.MESH,
            )
        pl.semaphore_wait(barrier_sem, 2)

        s = scale_ref[0, 0]

        def slot(origin):
            return pl.ds(lax.rem(origin + 2 * N_DEV, N_DEV) * m_per, m_per)

        def matmul_store(origin):
            chunk = xfull_ref[slot(origin), :]
            acc = lax.dot_general(
                chunk, w_ref[...],
                dimension_numbers=(((1,), (0,)), ((), ())),
                preferred_element_type=jnp.int32,
            )
            out_ref[slot(origin), :] = acc.astype(jnp.float32) * s

        def make_rdma(origin, dest, send_sem, recv_sem):
            return pltpu.make_async_remote_copy(
                src_ref=xfull_ref.at[slot(origin)],
                dst_ref=xfull_ref.at[slot(origin)],
                send_sem=send_sem,
                recv_sem=recv_sem,
                device_id=(dest,),
                device_id_type=pl.DeviceIdType.MESH,
            )

        r_rdma = [None] * R_HOPS
        l_rdma = [None] * L_HOPS

        xfull_ref[slot(my), :] = x_ref[...]
        r_rdma[0] = make_rdma(my, right, r_send_sems.at[0], r_recv_sems.at[0])
        r_rdma[0].start()
        l_rdma[0] = make_rdma(my, left, l_send_sems.at[0], l_recv_sems.at[0])
        l_rdma[0].start()
        matmul_store(my)

        for h in range(1, R_HOPS):
            r_rdma[h - 1].wait_recv()
            r_rdma[h] = make_rdma(my - h, right,
                                  r_send_sems.at[h], r_recv_sems.at[h])
            r_rdma[h].start()
            l_rdma[h - 1].wait_recv()
            if h < L_HOPS:
                l_rdma[h] = make_rdma(my + h, left,
                                      l_send_sems.at[h], l_recv_sems.at[h])
                l_rdma[h].start()
            matmul_store(my - h)
            matmul_store(my + h)

        r_rdma[R_HOPS - 1].wait_recv()
        matmul_store(my - R_HOPS)

        for h in range(R_HOPS):
            r_rdma[h].wait_send()
        for h in range(L_HOPS):
            l_rdma[h].wait_send()

    return pl.pallas_call(
        body,
        out_shape=jax.ShapeDtypeStruct((m_full, n_per), jnp.float32),
        in_specs=[
            pl.BlockSpec(memory_space=pltpu.VMEM),
            pl.BlockSpec(memory_space=pltpu.VMEM),
            pl.BlockSpec(memory_space=pltpu.SMEM),
        ],
        out_specs=pl.BlockSpec(memory_space=pltpu.VMEM),
        scratch_shapes=[
            pltpu.VMEM((m_full, k), jnp.int8),
            pltpu.SemaphoreType.DMA((R_HOPS,)),
            pltpu.SemaphoreType.DMA((R_HOPS,)),
            pltpu.SemaphoreType.DMA((L_HOPS,)),
            pltpu.SemaphoreType.DMA((L_HOPS,)),
        ],
        compiler_params=pltpu.CompilerParams(collective_id=0),
    )(x, w_mat, scale)


# device time: 96931 ns/iter; 2.2239x vs baseline; 1.1694x over previous
import jax
import jax.numpy as jnp
from jax import lax
from jax.experimental import pallas as pl
from jax.experimental.pallas import tpu as pltpu

N_DEV = 16
HOPS = N_DEV // 2


def kernel(x, w_mat, scale_x, scale_w):
    m_per, k = x.shape
    _, n_per = w_mat.shape
    m_full = N_DEV * m_per
    h_per = m_per // 2

    scale = (scale_x[0] * scale_w[0]).reshape(1, 1).astype(jnp.float32)

    def body(x_ref, w_ref, scale_ref, out_ref,
             xfull_ref, r_send_sems, r_recv_sems, l_send_sems, l_recv_sems):
        my = lax.axis_index("i")
        left = lax.rem(my + N_DEV - 1, N_DEV)
        right = lax.rem(my + 1, N_DEV)

        barrier_sem = pltpu.get_barrier_semaphore()
        for nbr in (left, right):
            pl.semaphore_signal(
                barrier_sem, inc=1,
                device_id=(nbr,), device_id_type=pl.DeviceIdType.MESH,
            )
        pl.semaphore_wait(barrier_sem, 2)

        s = scale_ref[0, 0]

        def chunk_row(origin):
            return lax.rem(origin + 2 * N_DEV, N_DEV) * m_per

        def half_slot(origin, half):
            return pl.ds(chunk_row(origin) + half * h_per, h_per)

        def matmul_store(origin, chunk=None):
            if chunk is None:
                chunk = xfull_ref[pl.ds(chunk_row(origin), m_per), :]
            acc = lax.dot_general(
                chunk, w_ref[...],
                dimension_numbers=(((1,), (0,)), ((), ())),
                preferred_element_type=jnp.int32,
            )
            out_ref[pl.ds(chunk_row(origin), m_per), :] = (
                acc.astype(jnp.float32) * s)

        def make_rdma(src, origin, half, dest, send_sem, recv_sem):
            return pltpu.make_async_remote_copy(
                src_ref=src,
                dst_ref=xfull_ref.at[half_slot(origin, half)],
                send_sem=send_sem,
                recv_sem=recv_sem,
                device_id=(dest,),
                device_id_type=pl.DeviceIdType.MESH,
            )

        r_rdma = [[None, None] for _ in range(HOPS)]
        l_rdma = [[None, None] for _ in range(HOPS)]

        for half in (0, 1):
            r_rdma[0][half] = make_rdma(
                x_ref.at[pl.ds(half * h_per, h_per)], my, half, right,
                r_send_sems.at[0, half], r_recv_sems.at[0, half])
            r_rdma[0][half].start()
            l_rdma[0][half] = make_rdma(
                x_ref.at[pl.ds(half * h_per, h_per)], my, half, left,
                l_send_sems.at[0, half], l_recv_sems.at[0, half])
            l_rdma[0][half].start()
        matmul_store(my, chunk=x_ref[...])

        for h in range(1, HOPS):
            last = h == HOPS - 1
            r_rdma[h - 1][0].wait_recv()
            r_rdma[h][0] = make_rdma(
                xfull_ref.at[half_slot(my - h, 0)], my - h, 0, right,
                r_send_sems.at[h, 0], r_recv_sems.at[h, 0])
            r_rdma[h][0].start()
            l_rdma[h - 1][1].wait_recv()
            l_rdma[h][1] = make_rdma(
                xfull_ref.at[half_slot(my + h, 1)], my + h, 1, left,
                l_send_sems.at[h, 1], l_recv_sems.at[h, 1])
            l_rdma[h][1].start()
            r_rdma[h - 1][1].wait_recv()
            if not last:
                r_rdma[h][1] = make_rdma(
                    xfull_ref.at[half_slot(my - h, 1)], my - h, 1, right,
                    r_send_sems.at[h, 1], r_recv_sems.at[h, 1])
                r_rdma[h][1].start()
            l_rdma[h - 1][0].wait_recv()
            if not last:
                l_rdma[h][0] = make_rdma(
                    xfull_ref.at[half_slot(my + h, 0)], my + h, 0, left,
                    l_send_sems.at[h, 0], l_recv_sems.at[h, 0])
                l_rdma[h][0].start()
            matmul_store(my - h)
            matmul_store(my + h)

        r_rdma[HOPS - 1][0].wait_recv()
        l_rdma[HOPS - 1][1].wait_recv()
        matmul_store(my - HOPS)

        for h in range(HOPS):
            for half in (0, 1):
                if r_rdma[h][half] is not None:
                    r_rdma[h][half].wait_send()
                if l_rdma[h][half] is not None:
                    l_rdma[h][half].wait_send()

    return pl.pallas_call(
        body,
        out_shape=jax.ShapeDtypeStruct((m_full, n_per), jnp.float32),
        in_specs=[
            pl.BlockSpec(memory_space=pltpu.VMEM),
            pl.BlockSpec(memory_space=pltpu.VMEM),
            pl.BlockSpec(memory_space=pltpu.SMEM),
        ],
        out_specs=pl.BlockSpec(memory_space=pltpu.VMEM),
        scratch_shapes=[
            pltpu.VMEM((m_full, k), jnp.int8),
            pltpu.SemaphoreType.DMA((HOPS, 2)),
            pltpu.SemaphoreType.DMA((HOPS, 2)),
            pltpu.SemaphoreType.DMA((HOPS, 2)),
            pltpu.SemaphoreType.DMA((HOPS, 2)),
        ],
        compiler_params=pltpu.CompilerParams(collective_id=0),
    )(x, w_mat, scale)
